# Initial kernel scaffold; baseline (speedup 1.0000x reference)
import jax
import jax.numpy as jnp
from jax.experimental import pallas as pl


def kernel(poi_embedding, img_embedding, mod_embed_table):
    raise NotImplementedError("write your pallas kernel here")


# speedup vs baseline: 4.3483x; 4.3483x over previous
import jax
import jax.numpy as jnp
from jax.experimental import pallas as pl

H = 128


def _add_row_kernel(row_idx, x_ref, tbl_ref, o_ref):
    o_ref[...] = x_ref[...] + tbl_ref[row_idx : row_idx + 1, :]


def _broadcast_add(x2d, table, row_idx, rows_per_blk):
    n_rows = x2d.shape[0]
    assert n_rows % rows_per_blk == 0
    grid = (n_rows // rows_per_blk,)
    return pl.pallas_call(
        lambda x_ref, t_ref, o_ref: _add_row_kernel(row_idx, x_ref, t_ref, o_ref),
        grid=grid,
        in_specs=[
            pl.BlockSpec((rows_per_blk, H), lambda i: (i, 0)),
            pl.BlockSpec((2, H), lambda i: (0, 0)),
        ],
        out_specs=pl.BlockSpec((rows_per_blk, H), lambda i: (i, 0)),
        out_shape=jax.ShapeDtypeStruct(x2d.shape, x2d.dtype),
    )(x2d, table)


def kernel(poi_embedding, img_embedding, mod_embed_table):
    B, S_poi, h = poi_embedding.shape
    S_img = img_embedding.shape[1]
    poi2 = poi_embedding.reshape(B * S_poi, h)
    img2 = img_embedding.reshape(B * S_img, h)
    poi_out = _broadcast_add(poi2, mod_embed_table, 0, 2048)
    img_out = _broadcast_add(img2, mod_embed_table, 1, 2048)
    return poi_out.reshape(B, S_poi, h), img_out.reshape(B, S_img, h)


# speedup vs baseline: 5.1599x; 1.1867x over previous
import jax
import jax.numpy as jnp
from jax.experimental import pallas as pl

H = 128


def _add_row_kernel(row_idx, x_ref, tbl_ref, o_ref):
    o_ref[...] = x_ref[...] + tbl_ref[row_idx : row_idx + 1, :]


def _broadcast_add(x2d, table, row_idx, rows_per_blk):
    n_rows = x2d.shape[0]
    assert n_rows % rows_per_blk == 0
    grid = (n_rows // rows_per_blk,)
    return pl.pallas_call(
        lambda x_ref, t_ref, o_ref: _add_row_kernel(row_idx, x_ref, t_ref, o_ref),
        grid=grid,
        in_specs=[
            pl.BlockSpec((rows_per_blk, H), lambda i: (i, 0)),
            pl.BlockSpec((2, H), lambda i: (0, 0)),
        ],
        out_specs=pl.BlockSpec((rows_per_blk, H), lambda i: (i, 0)),
        out_shape=jax.ShapeDtypeStruct(x2d.shape, x2d.dtype),
    )(x2d, table)


def kernel(poi_embedding, img_embedding, mod_embed_table):
    B, S_poi, h = poi_embedding.shape
    S_img = img_embedding.shape[1]
    poi2 = poi_embedding.reshape(B * S_poi, h)
    img2 = img_embedding.reshape(B * S_img, h)
    poi_out = _broadcast_add(poi2, mod_embed_table, 0, 8192)
    img_out = _broadcast_add(img2, mod_embed_table, 1, 6400)
    return poi_out.reshape(B, S_poi, h), img_out.reshape(B, S_img, h)


# speedup vs baseline: 5.1934x; 1.0065x over previous
import jax
import jax.numpy as jnp
from jax.experimental import pallas as pl

H = 128


def _add_row_kernel(row_idx, x_ref, tbl_ref, o_ref):
    o_ref[...] = x_ref[...] + tbl_ref[row_idx : row_idx + 1, :]


def _broadcast_add(x2d, table, row_idx, rows_per_blk):
    n_rows = x2d.shape[0]
    assert n_rows % rows_per_blk == 0
    grid = (n_rows // rows_per_blk,)
    return pl.pallas_call(
        lambda x_ref, t_ref, o_ref: _add_row_kernel(row_idx, x_ref, t_ref, o_ref),
        grid=grid,
        in_specs=[
            pl.BlockSpec((rows_per_blk, H), lambda i: (i, 0)),
            pl.BlockSpec((2, H), lambda i: (0, 0)),
        ],
        out_specs=pl.BlockSpec((rows_per_blk, H), lambda i: (i, 0)),
        out_shape=jax.ShapeDtypeStruct(x2d.shape, x2d.dtype),
    )(x2d, table)


def kernel(poi_embedding, img_embedding, mod_embed_table):
    B, S_poi, h = poi_embedding.shape
    S_img = img_embedding.shape[1]
    poi2 = poi_embedding.reshape(B * S_poi, h)
    img2 = img_embedding.reshape(B * S_img, h)
    poi_out = _broadcast_add(poi2, mod_embed_table, 0, 12800)
    img_out = _broadcast_add(img2, mod_embed_table, 1, 12800)
    return poi_out.reshape(B, S_poi, h), img_out.reshape(B, S_img, h)


# speedup vs baseline: 7.2534x; 1.3967x over previous
import jax
import jax.numpy as jnp
from jax.experimental import pallas as pl

H = 128


def _add_row_kernel(row_idx, x_ref, tbl_ref, o_ref):
    o_ref[...] = x_ref[...] + tbl_ref[row_idx : row_idx + 1, :][None]


def _broadcast_add(x, table, row_idx, batch_blk):
    B, S, h = x.shape
    assert B % batch_blk == 0
    grid = (B // batch_blk,)
    return pl.pallas_call(
        lambda x_ref, t_ref, o_ref: _add_row_kernel(row_idx, x_ref, t_ref, o_ref),
        grid=grid,
        in_specs=[
            pl.BlockSpec((batch_blk, S, h), lambda i: (i, 0, 0)),
            pl.BlockSpec((2, h), lambda i: (0, 0)),
        ],
        out_specs=pl.BlockSpec((batch_blk, S, h), lambda i: (i, 0, 0)),
        out_shape=jax.ShapeDtypeStruct(x.shape, x.dtype),
    )(x, table)


def kernel(poi_embedding, img_embedding, mod_embed_table):
    poi_out = _broadcast_add(poi_embedding, mod_embed_table, 0, 64)
    img_out = _broadcast_add(img_embedding, mod_embed_table, 1, 256)
    return poi_out, img_out


# speedup vs baseline: 7.3640x; 1.0152x over previous
import jax
import jax.numpy as jnp
from jax.experimental import pallas as pl

H = 128
BATCH_BLK = 64


def _modal_add_kernel(poi_ref, img_ref, tbl_ref, poi_out_ref, img_out_ref):
    poi_out_ref[...] = poi_ref[...] + tbl_ref[0:1, :][None]
    img_out_ref[...] = img_ref[...] + tbl_ref[1:2, :][None]


def kernel(poi_embedding, img_embedding, mod_embed_table):
    B, S_poi, h = poi_embedding.shape
    S_img = img_embedding.shape[1]
    grid = (B // BATCH_BLK,)
    return pl.pallas_call(
        _modal_add_kernel,
        grid=grid,
        in_specs=[
            pl.BlockSpec((BATCH_BLK, S_poi, h), lambda i: (i, 0, 0)),
            pl.BlockSpec((BATCH_BLK, S_img, h), lambda i: (i, 0, 0)),
            pl.BlockSpec((2, h), lambda i: (0, 0)),
        ],
        out_specs=[
            pl.BlockSpec((BATCH_BLK, S_poi, h), lambda i: (i, 0, 0)),
            pl.BlockSpec((BATCH_BLK, S_img, h), lambda i: (i, 0, 0)),
        ],
        out_shape=[
            jax.ShapeDtypeStruct(poi_embedding.shape, poi_embedding.dtype),
            jax.ShapeDtypeStruct(img_embedding.shape, img_embedding.dtype),
        ],
    )(poi_embedding, img_embedding, mod_embed_table)
